# baseline (speedup 1.0000x reference)
import jax
import jax.numpy as jnp
from jax.experimental import pallas as pl

PAD = 0.0


def _pad_mask_kernel(x_ref, out_ref, mask_ref):
    x = x_ref[...]
    out_ref[...] = x
    mask_ref[...] = x == PAD


def kernel(tokens_batch):
    B, L = tokens_batch.shape
    out, mask = pl.pallas_call(
        _pad_mask_kernel,
        out_shape=(
            jax.ShapeDtypeStruct((B, L), tokens_batch.dtype),
            jax.ShapeDtypeStruct((B, L), jnp.bool_),
        ),
    )(tokens_batch)
    return (out, mask)
